# Initial kernel scaffold; baseline (speedup 1.0000x reference)
import jax
import jax.numpy as jnp
from jax.experimental import pallas as pl


def kernel(atom_fea, nbr_fea, nbr_fea_idx, crystal_atom_idx, params):
    raise NotImplementedError("write your pallas kernel here")


# speedup vs baseline: 3.0744x; 3.0744x over previous
import jax
import jax.numpy as jnp
from jax.experimental import pallas as pl
from jax.experimental.pallas import tpu as pltpu
from jax.experimental.pallas import tpu_sc as plsc

F32 = jnp.float32

N_NBR = 12
ATOM_FEA_LEN = 64
NBR_FEA_LEN = 41
TWO_F = 2 * ATOM_FEA_LEN
EPS = 1e-5

_BA = 4000


def _sc_gather_rows(x, idx_flat, window):
    E = idx_flat.shape[1]
    C = x.shape[1]
    mesh = plsc.VectorSubcoreMesh(core_axis_name="core",
                                  subcore_axis_name="subcore")

    @pl.kernel(out_type=jax.ShapeDtypeStruct((E, C), x.dtype), mesh=mesh)
    def k(x_hbm, i_hbm, o_hbm):
        def body(i_vmem, o_vmem):
            pltpu.sync_copy(x_hbm.at[i_vmem.at[0]], o_vmem)

        pltpu.emit_pipeline(
            body,
            grid=(E // window,),
            in_specs=[pl.BlockSpec((1, window), lambda i: (0, i))],
            out_specs=[pl.BlockSpec((window, C), lambda i: (i, 0))],
            core_axis_name=("core", "subcore"),
            dimension_semantics=(pltpu.PARALLEL,),
        )(i_hbm, o_hbm)

    return k(x, idx_flat)


def _matmul_bias(x, W, b):
    n, kdim = x.shape
    m = W.shape[1]
    nb = n // _BA

    def body(x_ref, w_ref, b_ref, o_ref):
        o_ref[...] = (jnp.dot(x_ref[...], w_ref[...],
                              preferred_element_type=F32) + b_ref[...])

    return pl.pallas_call(
        body,
        grid=(nb,),
        in_specs=[
            pl.BlockSpec((_BA, kdim), lambda i: (i, 0)),
            pl.BlockSpec((kdim, m), lambda i: (0, 0)),
            pl.BlockSpec((1, m), lambda i: (0, 0)),
        ],
        out_specs=pl.BlockSpec((_BA, m), lambda i: (i, 0)),
        out_shape=jax.ShapeDtypeStruct((n, m), F32),
    )(x, W, b)


def _edge_kernel(x, gathered, fea_t, w_self, w_nbr, w_edge, bf):
    n = x.shape[0]
    nb = n // _BA

    def body(x_ref, g_ref, f_ref, ws_ref, wn_ref, we_ref, b_ref,
             gated_ref, s_ref, ss_ref, sself_ref):
        i = pl.program_id(0)
        m = pl.program_id(1)

        @pl.when(m == 0)
        def _():
            sself_ref[...] = (jnp.dot(x_ref[...], ws_ref[...],
                                      preferred_element_type=F32) + b_ref[...])

        g = (jnp.dot(g_ref[...], wn_ref[...], preferred_element_type=F32)
             + jnp.dot(f_ref[...], we_ref[...], preferred_element_type=F32)
             + sself_ref[...])
        gated_ref[...] = g

        @pl.when((i == 0) & (m == 0))
        def _():
            s_ref[...] = jnp.zeros_like(s_ref)
            ss_ref[...] = jnp.zeros_like(ss_ref)

        s_ref[...] += jnp.sum(g, axis=0, keepdims=True)
        ss_ref[...] += jnp.sum(g * g, axis=0, keepdims=True)

    return pl.pallas_call(
        body,
        grid=(nb, N_NBR),
        in_specs=[
            pl.BlockSpec((_BA, TWO_F), lambda i, m: (i, 0)),
            pl.BlockSpec((_BA, TWO_F), lambda i, m: (m * nb + i, 0)),
            pl.BlockSpec((_BA, NBR_FEA_LEN), lambda i, m: (m * nb + i, 0)),
            pl.BlockSpec((TWO_F, TWO_F), lambda i, m: (0, 0)),
            pl.BlockSpec((TWO_F, TWO_F), lambda i, m: (0, 0)),
            pl.BlockSpec((NBR_FEA_LEN, TWO_F), lambda i, m: (0, 0)),
            pl.BlockSpec((1, TWO_F), lambda i, m: (0, 0)),
        ],
        out_specs=[
            pl.BlockSpec((_BA, TWO_F), lambda i, m: (m * nb + i, 0)),
            pl.BlockSpec((1, TWO_F), lambda i, m: (0, 0)),
            pl.BlockSpec((1, TWO_F), lambda i, m: (0, 0)),
        ],
        out_shape=[
            jax.ShapeDtypeStruct((N_NBR * n, TWO_F), F32),
            jax.ShapeDtypeStruct((1, TWO_F), F32),
            jax.ShapeDtypeStruct((1, TWO_F), F32),
        ],
        scratch_shapes=[pltpu.VMEM((_BA, TWO_F), F32)],
    )(x, gathered, fea_t, w_self, w_nbr, w_edge, bf)


def _gate_kernel(gated, a1, c1, n):
    nb = n // _BA

    def body(g_ref, a_ref, c_ref, p_ref, s_ref, ss_ref, acc_ref):
        i = pl.program_id(0)
        m = pl.program_id(1)
        g = g_ref[...] * a_ref[...] + c_ref[...]
        filt = jax.nn.sigmoid(g[:, :ATOM_FEA_LEN])
        core = jax.nn.softplus(g[:, ATOM_FEA_LEN:])
        prod = filt * core

        @pl.when(m == 0)
        def _():
            acc_ref[...] = prod

        @pl.when(m > 0)
        def _():
            acc_ref[...] += prod

        @pl.when(m == N_NBR - 1)
        def _():
            p_ref[...] = acc_ref[...]

        @pl.when((i == 0) & (m == N_NBR - 1))
        def _():
            s_ref[...] = jnp.zeros_like(s_ref)
            ss_ref[...] = jnp.zeros_like(ss_ref)

        @pl.when(m == N_NBR - 1)
        def _():
            p = acc_ref[...]
            s_ref[...] += jnp.sum(p, axis=0, keepdims=True)
            ss_ref[...] += jnp.sum(p * p, axis=0, keepdims=True)

    return pl.pallas_call(
        body,
        grid=(nb, N_NBR),
        in_specs=[
            pl.BlockSpec((_BA, TWO_F), lambda i, m: (m * nb + i, 0)),
            pl.BlockSpec((1, TWO_F), lambda i, m: (0, 0)),
            pl.BlockSpec((1, TWO_F), lambda i, m: (0, 0)),
        ],
        out_specs=[
            pl.BlockSpec((_BA, ATOM_FEA_LEN), lambda i, m: (i, 0)),
            pl.BlockSpec((1, ATOM_FEA_LEN), lambda i, m: (0, 0)),
            pl.BlockSpec((1, ATOM_FEA_LEN), lambda i, m: (0, 0)),
        ],
        out_shape=[
            jax.ShapeDtypeStruct((n, ATOM_FEA_LEN), F32),
            jax.ShapeDtypeStruct((1, ATOM_FEA_LEN), F32),
            jax.ShapeDtypeStruct((1, ATOM_FEA_LEN), F32),
        ],
        scratch_shapes=[pltpu.VMEM((_BA, ATOM_FEA_LEN), F32)],
    )(gated, a1, c1)


def _resid_kernel(x, pre2, a2, c2):
    n = x.shape[0]
    nb = n // _BA

    def body(x_ref, p_ref, a_ref, c_ref, o_ref):
        res = jax.nn.softplus(
            x_ref[:, :ATOM_FEA_LEN] + p_ref[...] * a_ref[...] + c_ref[...])
        o_ref[...] = jnp.concatenate(
            [res, jnp.zeros_like(res)], axis=1)

    return pl.pallas_call(
        body,
        grid=(nb,),
        in_specs=[
            pl.BlockSpec((_BA, TWO_F), lambda i: (i, 0)),
            pl.BlockSpec((_BA, ATOM_FEA_LEN), lambda i: (i, 0)),
            pl.BlockSpec((1, ATOM_FEA_LEN), lambda i: (0, 0)),
            pl.BlockSpec((1, ATOM_FEA_LEN), lambda i: (0, 0)),
        ],
        out_specs=pl.BlockSpec((_BA, TWO_F), lambda i: (i, 0)),
        out_shape=jax.ShapeDtypeStruct((n, TWO_F), F32),
    )(x, pre2, a2, c2)


def _head_kernel(pooled, w_fc, b_fc, w_out, b_out, n_crystals, atoms_per):
    h = w_fc.shape[1]

    def body(p_ref, wf_ref, bf_ref, wo_ref, bo_ref, o_ref, acc_ref):
        j = pl.program_id(0)

        @pl.when(j == 0)
        def _():
            acc_ref[...] = p_ref[:, :ATOM_FEA_LEN]

        @pl.when(j > 0)
        def _():
            acc_ref[...] += p_ref[:, :ATOM_FEA_LEN]

        @pl.when(j == atoms_per - 1)
        def _():
            crys = acc_ref[...] * (1.0 / atoms_per)
            hid = jax.nn.softplus(
                jnp.dot(crys, wf_ref[...], preferred_element_type=F32)
                + bf_ref[...])
            o_ref[...] = (jnp.dot(hid, wo_ref[...],
                                  preferred_element_type=F32) + bo_ref[...])

    return pl.pallas_call(
        body,
        grid=(atoms_per,),
        in_specs=[
            pl.BlockSpec((n_crystals, TWO_F), lambda j: (j, 0)),
            pl.BlockSpec((ATOM_FEA_LEN, h), lambda j: (0, 0)),
            pl.BlockSpec((1, h), lambda j: (0, 0)),
            pl.BlockSpec((h, 1), lambda j: (0, 0)),
            pl.BlockSpec((1, 1), lambda j: (0, 0)),
        ],
        out_specs=pl.BlockSpec((n_crystals, 1), lambda j: (0, 0)),
        out_shape=jax.ShapeDtypeStruct((n_crystals, 1), F32),
        scratch_shapes=[pltpu.VMEM((n_crystals, ATOM_FEA_LEN), F32)],
    )(pooled, w_fc, b_fc, w_out, b_out)


def kernel(atom_fea, nbr_fea, nbr_fea_idx, crystal_atom_idx, params):
    p = params
    n, _ = atom_fea.shape
    n_edges = n * N_NBR
    n_crystals, atoms_per = crystal_atom_idx.shape

    w_emb = jnp.pad(p['W_emb'], ((0, 0), (0, ATOM_FEA_LEN)))
    b_emb = jnp.pad(p['b_emb'].reshape(1, -1), ((0, 0), (0, ATOM_FEA_LEN)))
    x = _matmul_bias(atom_fea, w_emb, b_emb)

    idx_t = nbr_fea_idx.astype(jnp.int32).T.reshape(1, n_edges)
    fea_t = jnp.transpose(nbr_fea, (1, 0, 2)).reshape(n_edges, NBR_FEA_LEN)

    for i in range(3):
        Wf = p['conv%d_Wf' % i]
        bf = p['conv%d_bf' % i].reshape(1, -1)
        gathered = _sc_gather_rows(x, idx_t, 128)
        w_self = jnp.pad(Wf[:ATOM_FEA_LEN], ((0, ATOM_FEA_LEN), (0, 0)))
        w_nbr = jnp.pad(Wf[ATOM_FEA_LEN:2 * ATOM_FEA_LEN],
                        ((0, ATOM_FEA_LEN), (0, 0)))
        gated, s1, ss1 = _edge_kernel(
            x, gathered, fea_t, w_self, w_nbr, Wf[2 * ATOM_FEA_LEN:], bf)
        mean1 = s1 / n_edges
        var1 = ss1 / n_edges - mean1 * mean1
        a1 = p['conv%d_bn1_g' % i].reshape(1, -1) * jax.lax.rsqrt(var1 + EPS)
        c1 = p['conv%d_bn1_b' % i].reshape(1, -1) - mean1 * a1
        pre2, s2, ss2 = _gate_kernel(gated, a1, c1, n)
        mean2 = s2 / n
        var2 = ss2 / n - mean2 * mean2
        a2 = p['conv%d_bn2_g' % i].reshape(1, -1) * jax.lax.rsqrt(var2 + EPS)
        c2 = p['conv%d_bn2_b' % i].reshape(1, -1) - mean2 * a2
        x = _resid_kernel(x, pre2, a2, c2)

    cidx_t = crystal_atom_idx.astype(jnp.int32).T.reshape(
        1, n_crystals * atoms_per)
    pooled = _sc_gather_rows(x, cidx_t, 128)
    return _head_kernel(pooled, p['W_fc'], p['b_fc'].reshape(1, -1),
                        p['W_out'], p['b_out'].reshape(1, 1),
                        n_crystals, atoms_per)


# speedup vs baseline: 4.3614x; 1.4186x over previous
import jax
import jax.numpy as jnp
from jax.experimental import pallas as pl
from jax.experimental.pallas import tpu as pltpu
from jax.experimental.pallas import tpu_sc as plsc

F32 = jnp.float32

N_NBR = 12
ATOM_FEA_LEN = 64
NBR_FEA_LEN = 41
TWO_F = 2 * ATOM_FEA_LEN
EPS = 1e-5

_BA = 4000


def _sc_gather_rows(x, idx_flat, window):
    E = idx_flat.shape[1]
    C = x.shape[1]
    mesh = plsc.VectorSubcoreMesh(core_axis_name="core",
                                  subcore_axis_name="subcore")

    @pl.kernel(out_type=jax.ShapeDtypeStruct((E, C), x.dtype), mesh=mesh)
    def k(x_hbm, i_hbm, o_hbm):
        def body(i_vmem, o_vmem):
            pltpu.sync_copy(x_hbm.at[i_vmem.at[0]], o_vmem)

        pltpu.emit_pipeline(
            body,
            grid=(E // window,),
            in_specs=[pl.BlockSpec((1, window), lambda i: (0, i))],
            out_specs=[pl.BlockSpec((window, C), lambda i: (i, 0))],
            core_axis_name=("core", "subcore"),
            dimension_semantics=(pltpu.PARALLEL,),
        )(i_hbm, o_hbm)

    return k(x, idx_flat)


def _emb_kernel(x, W, b):
    n, kdim = x.shape
    m = W.shape[1]
    nb = n // _BA

    def body(x_ref, w_ref, b_ref, y_ref, p_ref):
        y = (jnp.dot(x_ref[...], w_ref[...],
                     preferred_element_type=F32) + b_ref[...])
        y_ref[...] = y
        p_ref[...] = jnp.concatenate([y, jnp.zeros_like(y)], axis=1)

    return pl.pallas_call(
        body,
        grid=(nb,),
        in_specs=[
            pl.BlockSpec((_BA, kdim), lambda i: (i, 0)),
            pl.BlockSpec((kdim, m), lambda i: (0, 0)),
            pl.BlockSpec((1, m), lambda i: (0, 0)),
        ],
        out_specs=[
            pl.BlockSpec((_BA, m), lambda i: (i, 0)),
            pl.BlockSpec((_BA, 2 * m), lambda i: (i, 0)),
        ],
        out_shape=[
            jax.ShapeDtypeStruct((n, m), F32),
            jax.ShapeDtypeStruct((n, 2 * m), F32),
        ],
    )(x, W, b)


def _edge_kernel(x, gathered, fea_t, w_self, w_nbr, w_edge, bf):
    n = x.shape[0]
    nb = n // _BA

    def body(x_ref, g_ref, f_ref, ws_ref, wn_ref, we_ref, b_ref,
             gated_ref, s_ref, ss_ref, sself_ref):
        i = pl.program_id(0)
        m = pl.program_id(1)

        @pl.when(m == 0)
        def _():
            sself_ref[...] = (jnp.dot(x_ref[...], ws_ref[...],
                                      preferred_element_type=F32) + b_ref[...])

        g = (jnp.dot(g_ref[...], wn_ref[...], preferred_element_type=F32)
             + jnp.dot(f_ref[...], we_ref[...], preferred_element_type=F32)
             + sself_ref[...])
        gated_ref[...] = g

        @pl.when((i == 0) & (m == 0))
        def _():
            s_ref[...] = jnp.zeros_like(s_ref)
            ss_ref[...] = jnp.zeros_like(ss_ref)

        s_ref[...] += jnp.sum(g, axis=0, keepdims=True)
        ss_ref[...] += jnp.sum(g * g, axis=0, keepdims=True)

    return pl.pallas_call(
        body,
        grid=(nb, N_NBR),
        in_specs=[
            pl.BlockSpec((_BA, ATOM_FEA_LEN), lambda i, m: (i, 0)),
            pl.BlockSpec((_BA, TWO_F), lambda i, m: (m * nb + i, 0)),
            pl.BlockSpec((_BA, NBR_FEA_LEN), lambda i, m: (m * nb + i, 0)),
            pl.BlockSpec((ATOM_FEA_LEN, TWO_F), lambda i, m: (0, 0)),
            pl.BlockSpec((TWO_F, TWO_F), lambda i, m: (0, 0)),
            pl.BlockSpec((NBR_FEA_LEN, TWO_F), lambda i, m: (0, 0)),
            pl.BlockSpec((1, TWO_F), lambda i, m: (0, 0)),
        ],
        out_specs=[
            pl.BlockSpec((_BA, TWO_F), lambda i, m: (m * nb + i, 0)),
            pl.BlockSpec((1, TWO_F), lambda i, m: (0, 0)),
            pl.BlockSpec((1, TWO_F), lambda i, m: (0, 0)),
        ],
        out_shape=[
            jax.ShapeDtypeStruct((N_NBR * n, TWO_F), F32),
            jax.ShapeDtypeStruct((1, TWO_F), F32),
            jax.ShapeDtypeStruct((1, TWO_F), F32),
        ],
        scratch_shapes=[pltpu.VMEM((_BA, TWO_F), F32)],
    )(x, gathered, fea_t, w_self, w_nbr, w_edge, bf)


def _gate_kernel(gated, a1, c1, n):
    nb = n // _BA

    def body(g_ref, a_ref, c_ref, p_ref, s_ref, ss_ref, acc_ref):
        i = pl.program_id(0)
        m = pl.program_id(1)
        g = g_ref[...].astype(F32) * a_ref[...] + c_ref[...]
        filt = jax.nn.sigmoid(g[:, :ATOM_FEA_LEN])
        core = jax.nn.softplus(g[:, ATOM_FEA_LEN:])
        prod = filt * core

        @pl.when(m == 0)
        def _():
            acc_ref[...] = prod

        @pl.when(m > 0)
        def _():
            acc_ref[...] += prod

        @pl.when(m == N_NBR - 1)
        def _():
            p_ref[...] = acc_ref[...]

        @pl.when((i == 0) & (m == N_NBR - 1))
        def _():
            s_ref[...] = jnp.zeros_like(s_ref)
            ss_ref[...] = jnp.zeros_like(ss_ref)

        @pl.when(m == N_NBR - 1)
        def _():
            p = acc_ref[...]
            s_ref[...] += jnp.sum(p, axis=0, keepdims=True)
            ss_ref[...] += jnp.sum(p * p, axis=0, keepdims=True)

    return pl.pallas_call(
        body,
        grid=(nb, N_NBR),
        in_specs=[
            pl.BlockSpec((_BA, TWO_F), lambda i, m: (m * nb + i, 0)),
            pl.BlockSpec((1, TWO_F), lambda i, m: (0, 0)),
            pl.BlockSpec((1, TWO_F), lambda i, m: (0, 0)),
        ],
        out_specs=[
            pl.BlockSpec((_BA, ATOM_FEA_LEN), lambda i, m: (i, 0)),
            pl.BlockSpec((1, ATOM_FEA_LEN), lambda i, m: (0, 0)),
            pl.BlockSpec((1, ATOM_FEA_LEN), lambda i, m: (0, 0)),
        ],
        out_shape=[
            jax.ShapeDtypeStruct((n, ATOM_FEA_LEN), F32),
            jax.ShapeDtypeStruct((1, ATOM_FEA_LEN), F32),
            jax.ShapeDtypeStruct((1, ATOM_FEA_LEN), F32),
        ],
        scratch_shapes=[pltpu.VMEM((_BA, ATOM_FEA_LEN), F32)],
    )(gated, a1, c1)


def _resid_kernel(x, pre2, a2, c2):
    n = x.shape[0]
    nb = n // _BA

    def body(x_ref, p_ref, a_ref, c_ref, y_ref, o_ref):
        res = jax.nn.softplus(
            x_ref[...] + p_ref[...] * a_ref[...] + c_ref[...])
        y_ref[...] = res
        o_ref[...] = jnp.concatenate([res, jnp.zeros_like(res)], axis=1)

    return pl.pallas_call(
        body,
        grid=(nb,),
        in_specs=[
            pl.BlockSpec((_BA, ATOM_FEA_LEN), lambda i: (i, 0)),
            pl.BlockSpec((_BA, ATOM_FEA_LEN), lambda i: (i, 0)),
            pl.BlockSpec((1, ATOM_FEA_LEN), lambda i: (0, 0)),
            pl.BlockSpec((1, ATOM_FEA_LEN), lambda i: (0, 0)),
        ],
        out_specs=[
            pl.BlockSpec((_BA, ATOM_FEA_LEN), lambda i: (i, 0)),
            pl.BlockSpec((_BA, TWO_F), lambda i: (i, 0)),
        ],
        out_shape=[
            jax.ShapeDtypeStruct((n, ATOM_FEA_LEN), F32),
            jax.ShapeDtypeStruct((n, TWO_F), F32),
        ],
    )(x, pre2, a2, c2)


def _head_kernel(pooled, w_fc, b_fc, w_out, b_out, n_crystals, atoms_per):
    h = w_fc.shape[1]

    def body(p_ref, wf_ref, bf_ref, wo_ref, bo_ref, o_ref, acc_ref):
        j = pl.program_id(0)

        @pl.when(j == 0)
        def _():
            acc_ref[...] = p_ref[...]

        @pl.when(j > 0)
        def _():
            acc_ref[...] += p_ref[...]

        @pl.when(j == atoms_per - 1)
        def _():
            crys = acc_ref[...] * (1.0 / atoms_per)
            hid = jax.nn.softplus(
                jnp.dot(crys, wf_ref[...], preferred_element_type=F32)
                + bf_ref[...])
            o_ref[...] = (jnp.dot(hid, wo_ref[...],
                                  preferred_element_type=F32) + bo_ref[...])

    return pl.pallas_call(
        body,
        grid=(atoms_per,),
        in_specs=[
            pl.BlockSpec((n_crystals, TWO_F), lambda j: (j, 0)),
            pl.BlockSpec((TWO_F, h), lambda j: (0, 0)),
            pl.BlockSpec((1, h), lambda j: (0, 0)),
            pl.BlockSpec((h, 1), lambda j: (0, 0)),
            pl.BlockSpec((1, 1), lambda j: (0, 0)),
        ],
        out_specs=pl.BlockSpec((n_crystals, 1), lambda j: (0, 0)),
        out_shape=jax.ShapeDtypeStruct((n_crystals, 1), F32),
        scratch_shapes=[pltpu.VMEM((n_crystals, TWO_F), F32)],
    )(pooled, w_fc, b_fc, w_out, b_out)


def kernel(atom_fea, nbr_fea, nbr_fea_idx, crystal_atom_idx, params):
    p = params
    n, _ = atom_fea.shape
    n_edges = n * N_NBR
    n_crystals, atoms_per = crystal_atom_idx.shape

    x, x_pad = _emb_kernel(atom_fea, p['W_emb'], p['b_emb'].reshape(1, -1))

    idx_t = nbr_fea_idx.astype(jnp.int32).T.reshape(1, n_edges)
    fea_t = jnp.transpose(nbr_fea, (1, 0, 2)).reshape(n_edges, NBR_FEA_LEN)

    for i in range(3):
        Wf = p['conv%d_Wf' % i]
        bf = p['conv%d_bf' % i].reshape(1, -1)
        gathered = _sc_gather_rows(x_pad, idx_t, 128)
        w_self = Wf[:ATOM_FEA_LEN]
        w_nbr = jnp.pad(Wf[ATOM_FEA_LEN:2 * ATOM_FEA_LEN],
                        ((0, ATOM_FEA_LEN), (0, 0)))
        w_edge = Wf[2 * ATOM_FEA_LEN:]
        gated, s1, ss1 = _edge_kernel(
            x, gathered, fea_t, w_self, w_nbr, w_edge, bf)
        mean1 = s1 / n_edges
        var1 = ss1 / n_edges - mean1 * mean1
        a1 = p['conv%d_bn1_g' % i].reshape(1, -1) * jax.lax.rsqrt(var1 + EPS)
        c1 = p['conv%d_bn1_b' % i].reshape(1, -1) - mean1 * a1
        pre2, s2, ss2 = _gate_kernel(gated, a1, c1, n)
        mean2 = s2 / n
        var2 = ss2 / n - mean2 * mean2
        a2 = p['conv%d_bn2_g' % i].reshape(1, -1) * jax.lax.rsqrt(var2 + EPS)
        c2 = p['conv%d_bn2_b' % i].reshape(1, -1) - mean2 * a2
        x, x_pad = _resid_kernel(x, pre2, a2, c2)

    cidx_t = crystal_atom_idx.astype(jnp.int32).T.reshape(
        1, n_crystals * atoms_per)
    pooled = _sc_gather_rows(x_pad, cidx_t, 128)
    w_fc_pad = jnp.pad(p['W_fc'], ((0, ATOM_FEA_LEN), (0, 0)))
    return _head_kernel(pooled, w_fc_pad, p['b_fc'].reshape(1, -1),
                        p['W_out'], p['b_out'].reshape(1, 1),
                        n_crystals, atoms_per)


# speedup vs baseline: 4.3623x; 1.0002x over previous
import jax
import jax.numpy as jnp
from jax.experimental import pallas as pl
from jax.experimental.pallas import tpu as pltpu
from jax.experimental.pallas import tpu_sc as plsc

F32 = jnp.float32

N_NBR = 12
ATOM_FEA_LEN = 64
NBR_FEA_LEN = 41
TWO_F = 2 * ATOM_FEA_LEN
EPS = 1e-5

_BA = 4000


def _sc_gather_rows(x, idx_flat, window):
    E = idx_flat.shape[1]
    C = x.shape[1]
    mesh = plsc.VectorSubcoreMesh(core_axis_name="core",
                                  subcore_axis_name="subcore")

    @pl.kernel(out_type=jax.ShapeDtypeStruct((E, C), x.dtype), mesh=mesh)
    def k(x_hbm, i_hbm, o_hbm):
        def body(i_vmem, o_vmem):
            pltpu.sync_copy(x_hbm.at[i_vmem.at[0]], o_vmem)

        pltpu.emit_pipeline(
            body,
            grid=(E // window,),
            in_specs=[pl.BlockSpec((1, window), lambda i: (0, i))],
            out_specs=[pl.BlockSpec((window, C), lambda i: (i, 0))],
            core_axis_name=("core", "subcore"),
            dimension_semantics=(pltpu.PARALLEL,),
        )(i_hbm, o_hbm)

    return k(x, idx_flat)


def _emb_kernel(x, W, b):
    n, kdim = x.shape
    m = W.shape[1]
    nb = n // _BA

    def body(x_ref, w_ref, b_ref, y_ref, p_ref):
        y = (jnp.dot(x_ref[...], w_ref[...],
                     preferred_element_type=F32) + b_ref[...])
        y_ref[...] = y
        p_ref[...] = jnp.concatenate([y, jnp.zeros_like(y)], axis=1)

    return pl.pallas_call(
        body,
        grid=(nb,),
        in_specs=[
            pl.BlockSpec((_BA, kdim), lambda i: (i, 0)),
            pl.BlockSpec((kdim, m), lambda i: (0, 0)),
            pl.BlockSpec((1, m), lambda i: (0, 0)),
        ],
        out_specs=[
            pl.BlockSpec((_BA, m), lambda i: (i, 0)),
            pl.BlockSpec((_BA, 2 * m), lambda i: (i, 0)),
        ],
        out_shape=[
            jax.ShapeDtypeStruct((n, m), F32),
            jax.ShapeDtypeStruct((n, 2 * m), F32),
        ],
    )(x, W, b)


def _proj_kernel(x, w_nbr):
    n = x.shape[0]
    nb = n // _BA

    def body(x_ref, w_ref, z_ref):
        z_ref[...] = jnp.dot(x_ref[...], w_ref[...],
                             preferred_element_type=F32)

    return pl.pallas_call(
        body,
        grid=(nb,),
        in_specs=[
            pl.BlockSpec((_BA, ATOM_FEA_LEN), lambda i: (i, 0)),
            pl.BlockSpec((ATOM_FEA_LEN, TWO_F), lambda i: (0, 0)),
        ],
        out_specs=pl.BlockSpec((_BA, TWO_F), lambda i: (i, 0)),
        out_shape=jax.ShapeDtypeStruct((n, TWO_F), F32),
    )(x, w_nbr)


def _edge_kernel(x, gathered, fea_t, w_self, w_edge, bf):
    n = x.shape[0]
    nb = n // _BA

    def body(x_ref, g_ref, f_ref, ws_ref, we_ref, b_ref,
             gated_ref, s_ref, ss_ref, sself_ref):
        i = pl.program_id(0)
        m = pl.program_id(1)

        @pl.when(m == 0)
        def _():
            sself_ref[...] = (jnp.dot(x_ref[...], ws_ref[...],
                                      preferred_element_type=F32) + b_ref[...])

        g = (g_ref[...]
             + jnp.dot(f_ref[...], we_ref[...], preferred_element_type=F32)
             + sself_ref[...])
        gated_ref[...] = g

        @pl.when((i == 0) & (m == 0))
        def _():
            s_ref[...] = jnp.zeros_like(s_ref)
            ss_ref[...] = jnp.zeros_like(ss_ref)

        s_ref[...] += jnp.sum(g, axis=0, keepdims=True)
        ss_ref[...] += jnp.sum(g * g, axis=0, keepdims=True)

    return pl.pallas_call(
        body,
        grid=(nb, N_NBR),
        in_specs=[
            pl.BlockSpec((_BA, ATOM_FEA_LEN), lambda i, m: (i, 0)),
            pl.BlockSpec((_BA, TWO_F), lambda i, m: (m * nb + i, 0)),
            pl.BlockSpec((_BA, NBR_FEA_LEN), lambda i, m: (m * nb + i, 0)),
            pl.BlockSpec((ATOM_FEA_LEN, TWO_F), lambda i, m: (0, 0)),
            pl.BlockSpec((NBR_FEA_LEN, TWO_F), lambda i, m: (0, 0)),
            pl.BlockSpec((1, TWO_F), lambda i, m: (0, 0)),
        ],
        out_specs=[
            pl.BlockSpec((_BA, TWO_F), lambda i, m: (m * nb + i, 0)),
            pl.BlockSpec((1, TWO_F), lambda i, m: (0, 0)),
            pl.BlockSpec((1, TWO_F), lambda i, m: (0, 0)),
        ],
        out_shape=[
            jax.ShapeDtypeStruct((N_NBR * n, TWO_F), F32),
            jax.ShapeDtypeStruct((1, TWO_F), F32),
            jax.ShapeDtypeStruct((1, TWO_F), F32),
        ],
        scratch_shapes=[pltpu.VMEM((_BA, TWO_F), F32)],
    )(x, gathered, fea_t, w_self, w_edge, bf)


def _gate_kernel(gated, a1, c1, n):
    nb = n // _BA

    def body(g_ref, a_ref, c_ref, p_ref, s_ref, ss_ref, acc_ref):
        i = pl.program_id(0)
        m = pl.program_id(1)
        g = g_ref[...].astype(F32) * a_ref[...] + c_ref[...]
        filt = jax.nn.sigmoid(g[:, :ATOM_FEA_LEN])
        core = jax.nn.softplus(g[:, ATOM_FEA_LEN:])
        prod = filt * core

        @pl.when(m == 0)
        def _():
            acc_ref[...] = prod

        @pl.when(m > 0)
        def _():
            acc_ref[...] += prod

        @pl.when(m == N_NBR - 1)
        def _():
            p_ref[...] = acc_ref[...]

        @pl.when((i == 0) & (m == N_NBR - 1))
        def _():
            s_ref[...] = jnp.zeros_like(s_ref)
            ss_ref[...] = jnp.zeros_like(ss_ref)

        @pl.when(m == N_NBR - 1)
        def _():
            p = acc_ref[...]
            s_ref[...] += jnp.sum(p, axis=0, keepdims=True)
            ss_ref[...] += jnp.sum(p * p, axis=0, keepdims=True)

    return pl.pallas_call(
        body,
        grid=(nb, N_NBR),
        in_specs=[
            pl.BlockSpec((_BA, TWO_F), lambda i, m: (m * nb + i, 0)),
            pl.BlockSpec((1, TWO_F), lambda i, m: (0, 0)),
            pl.BlockSpec((1, TWO_F), lambda i, m: (0, 0)),
        ],
        out_specs=[
            pl.BlockSpec((_BA, ATOM_FEA_LEN), lambda i, m: (i, 0)),
            pl.BlockSpec((1, ATOM_FEA_LEN), lambda i, m: (0, 0)),
            pl.BlockSpec((1, ATOM_FEA_LEN), lambda i, m: (0, 0)),
        ],
        out_shape=[
            jax.ShapeDtypeStruct((n, ATOM_FEA_LEN), F32),
            jax.ShapeDtypeStruct((1, ATOM_FEA_LEN), F32),
            jax.ShapeDtypeStruct((1, ATOM_FEA_LEN), F32),
        ],
        scratch_shapes=[pltpu.VMEM((_BA, ATOM_FEA_LEN), F32)],
    )(gated, a1, c1)


def _resid_kernel(x, pre2, a2, c2):
    n = x.shape[0]
    nb = n // _BA

    def body(x_ref, p_ref, a_ref, c_ref, y_ref, o_ref):
        res = jax.nn.softplus(
            x_ref[...] + p_ref[...] * a_ref[...] + c_ref[...])
        y_ref[...] = res
        o_ref[...] = jnp.concatenate([res, jnp.zeros_like(res)], axis=1)

    return pl.pallas_call(
        body,
        grid=(nb,),
        in_specs=[
            pl.BlockSpec((_BA, ATOM_FEA_LEN), lambda i: (i, 0)),
            pl.BlockSpec((_BA, ATOM_FEA_LEN), lambda i: (i, 0)),
            pl.BlockSpec((1, ATOM_FEA_LEN), lambda i: (0, 0)),
            pl.BlockSpec((1, ATOM_FEA_LEN), lambda i: (0, 0)),
        ],
        out_specs=[
            pl.BlockSpec((_BA, ATOM_FEA_LEN), lambda i: (i, 0)),
            pl.BlockSpec((_BA, TWO_F), lambda i: (i, 0)),
        ],
        out_shape=[
            jax.ShapeDtypeStruct((n, ATOM_FEA_LEN), F32),
            jax.ShapeDtypeStruct((n, TWO_F), F32),
        ],
    )(x, pre2, a2, c2)


def _head_kernel(pooled, w_fc, b_fc, w_out, b_out, n_crystals, atoms_per):
    h = w_fc.shape[1]

    def body(p_ref, wf_ref, bf_ref, wo_ref, bo_ref, o_ref, acc_ref):
        j = pl.program_id(0)

        @pl.when(j == 0)
        def _():
            acc_ref[...] = p_ref[...]

        @pl.when(j > 0)
        def _():
            acc_ref[...] += p_ref[...]

        @pl.when(j == atoms_per - 1)
        def _():
            crys = acc_ref[...] * (1.0 / atoms_per)
            hid = jax.nn.softplus(
                jnp.dot(crys, wf_ref[...], preferred_element_type=F32)
                + bf_ref[...])
            o_ref[...] = (jnp.dot(hid, wo_ref[...],
                                  preferred_element_type=F32) + bo_ref[...])

    return pl.pallas_call(
        body,
        grid=(atoms_per,),
        in_specs=[
            pl.BlockSpec((n_crystals, TWO_F), lambda j: (j, 0)),
            pl.BlockSpec((TWO_F, h), lambda j: (0, 0)),
            pl.BlockSpec((1, h), lambda j: (0, 0)),
            pl.BlockSpec((h, 1), lambda j: (0, 0)),
            pl.BlockSpec((1, 1), lambda j: (0, 0)),
        ],
        out_specs=pl.BlockSpec((n_crystals, 1), lambda j: (0, 0)),
        out_shape=jax.ShapeDtypeStruct((n_crystals, 1), F32),
        scratch_shapes=[pltpu.VMEM((n_crystals, TWO_F), F32)],
    )(pooled, w_fc, b_fc, w_out, b_out)


def kernel(atom_fea, nbr_fea, nbr_fea_idx, crystal_atom_idx, params):
    p = params
    n, _ = atom_fea.shape
    n_edges = n * N_NBR
    n_crystals, atoms_per = crystal_atom_idx.shape

    x, x_pad = _emb_kernel(atom_fea, p['W_emb'], p['b_emb'].reshape(1, -1))

    idx_t = nbr_fea_idx.astype(jnp.int32).T.reshape(1, n_edges)
    fea_t = jnp.transpose(nbr_fea, (1, 0, 2)).reshape(n_edges, NBR_FEA_LEN)

    for i in range(3):
        Wf = p['conv%d_Wf' % i]
        bf = p['conv%d_bf' % i].reshape(1, -1)
        w_self = Wf[:ATOM_FEA_LEN]
        w_nbr = Wf[ATOM_FEA_LEN:2 * ATOM_FEA_LEN]
        w_edge = Wf[2 * ATOM_FEA_LEN:]
        z = _proj_kernel(x, w_nbr)
        gathered = _sc_gather_rows(z, idx_t, 128)
        gated, s1, ss1 = _edge_kernel(
            x, gathered, fea_t, w_self, w_edge, bf)
        mean1 = s1 / n_edges
        var1 = ss1 / n_edges - mean1 * mean1
        a1 = p['conv%d_bn1_g' % i].reshape(1, -1) * jax.lax.rsqrt(var1 + EPS)
        c1 = p['conv%d_bn1_b' % i].reshape(1, -1) - mean1 * a1
        pre2, s2, ss2 = _gate_kernel(gated, a1, c1, n)
        mean2 = s2 / n
        var2 = ss2 / n - mean2 * mean2
        a2 = p['conv%d_bn2_g' % i].reshape(1, -1) * jax.lax.rsqrt(var2 + EPS)
        c2 = p['conv%d_bn2_b' % i].reshape(1, -1) - mean2 * a2
        x, x_pad = _resid_kernel(x, pre2, a2, c2)

    cidx_t = crystal_atom_idx.astype(jnp.int32).T.reshape(
        1, n_crystals * atoms_per)
    pooled = _sc_gather_rows(x_pad, cidx_t, 128)
    w_fc_pad = jnp.pad(p['W_fc'], ((0, ATOM_FEA_LEN), (0, 0)))
    return _head_kernel(pooled, w_fc_pad, p['b_fc'].reshape(1, -1),
                        p['W_out'], p['b_out'].reshape(1, 1),
                        n_crystals, atoms_per)
